# baseline (speedup 1.0000x reference)
import functools

import jax
import jax.numpy as jnp
from jax import lax
from jax.experimental import pallas as pl
from jax.experimental.pallas import tpu as pltpu
from jax.experimental.pallas import tpu_sc as plsc

N_PULSES = 320000
NUM_DOMS = 10000
EMBED_DIM = 128
NLANE = 16
NREG = EMBED_DIM // NLANE

NC, NS = 2, 16
NW = NC * NS
DPW = 8 * -(-NUM_DOMS // (NW * 8))
NDP = DPW * NW
CHUNK = 128
NBLK = N_PULSES // 16
SEARCH_ITERS = 15


def _sc_pool(emb, idx):
  mesh = plsc.VectorSubcoreMesh(core_axis_name="c", subcore_axis_name="s")

  @functools.partial(
      pl.kernel,
      mesh=mesh,
      out_type=[
          jax.ShapeDtypeStruct((NDP * EMBED_DIM,), jnp.float32),
          jax.ShapeDtypeStruct((NDP * EMBED_DIM,), jnp.float32),
      ],
      scratch_types=[
          pltpu.VMEM((CHUNK * EMBED_DIM,), jnp.float32),
          pltpu.VMEM((CHUNK * EMBED_DIM,), jnp.float32),
          pltpu.VMEM((CHUNK,), jnp.int32),
          pltpu.VMEM((CHUNK,), jnp.int32),
          pltpu.VMEM((DPW * EMBED_DIM,), jnp.float32),
          pltpu.VMEM((DPW * EMBED_DIM,), jnp.float32),
          pltpu.VMEM((16,), jnp.int32),
          pltpu.VMEM((2 * EMBED_DIM,), jnp.float32),
          pltpu.SemaphoreType.DMA,
          pltpu.SemaphoreType.DMA,
      ],
  )
  def pool_kernel(emb_hbm, idx_hbm, mean_hbm, max_hbm,
                  rowbuf_a, rowbuf_b, idxbuf_a, idxbuf_b,
                  meanbuf, maxbuf, sbuf, accbuf, sem_a, sem_b):
    wid = lax.axis_index("s") * NC + lax.axis_index("c")
    d0 = wid * DPW
    d1 = jnp.minimum(d0 + DPW, NUM_DOMS)

    zeros = jnp.zeros((NLANE,), jnp.float32)

    def zero_row(r, carry):
      for j in range(NREG):
        meanbuf[pl.ds(r * EMBED_DIM + j * NLANE, NLANE)] = zeros
        maxbuf[pl.ds(r * EMBED_DIM + j * NLANE, NLANE)] = zeros
      return carry
    lax.fori_loop(0, DPW, zero_row, 0)

    def lower_bound(target):
      def it(_, c):
        lo, hi = c
        run = lo < hi
        mid = jnp.where(run, (lo + hi) // 2, 0)
        pltpu.sync_copy(idx_hbm.at[pl.ds(mid * 16, 16)], sbuf)
        probe = sbuf[pl.ds(0, NLANE)]
        ge = probe[0] >= target
        nlo = jnp.where(ge, lo, mid + 1)
        nhi = jnp.where(ge, mid, hi)
        return (jnp.where(run, nlo, lo), jnp.where(run, nhi, hi))
      lo, _ = lax.fori_loop(0, SEARCH_ITERS, it,
                            (jnp.int32(0), jnp.int32(NBLK)))
      return lo

    f0 = lower_bound(d0)
    f1 = lower_bound(d1)
    p0 = jnp.maximum(f0 - 1, 0) * 16
    p1 = f1 * 16
    c0 = p0 // CHUNK
    nch = (p1 + CHUNK - 1) // CHUNK - c0

    def flush(pred, cur, cnt, s, m):
      def fbody(i, z):
        off = (cur - d0) * EMBED_DIM
        cnt_v = jnp.broadcast_to(cnt, (NLANE,))
        for j in range(NREG):
          meanbuf[pl.ds(off + j * NLANE, NLANE)] = s[j] / cnt_v
          maxbuf[pl.ds(off + j * NLANE, NLANE)] = m[j]
        return z
      lax.fori_loop(0, pred.astype(jnp.int32), fbody, jnp.int32(0))

    NB2 = CHUNK * EMBED_DIM

    def issue(g, rb, ib, sem):
      @pl.when(g < nch)
      def _():
        base = (c0 + g) * CHUNK
        pltpu.make_async_copy(
            emb_hbm.at[pl.ds(base * EMBED_DIM, NB2)], rb, sem).start()
        pltpu.make_async_copy(idx_hbm.at[pl.ds(base, CHUNK)], ib, sem).start()

    def wait(g, rb, ib, sem):
      @pl.when(g < nch)
      def _():
        pltpu.make_async_copy(emb_hbm.at[pl.ds(0, NB2)], rb, sem).wait()
        pltpu.make_async_copy(idx_hbm.at[pl.ds(0, CHUNK)], ib, sem).wait()

    def load_acc():
      sv = tuple(accbuf[pl.ds(j * NLANE, NLANE)] for j in range(NREG))
      mv = tuple(accbuf[pl.ds(EMBED_DIM + j * NLANE, NLANE)]
                 for j in range(NREG))
      return sv, mv

    def store_acc(s, m):
      for j in range(NREG):
        accbuf[pl.ds(j * NLANE, NLANE)] = s[j]
        accbuf[pl.ds(EMBED_DIM + j * NLANE, NLANE)] = m[j]

    def process(rb, ib, cur0, cnt0):
      def rows_fast(row0, n, cur, cnt):
        s, m = load_acc()
        for k in range(n):
          roff = (row0 + k) * EMBED_DIM
          row = tuple(rb[pl.ds(roff + j * NLANE, NLANE)]
                      for j in range(NREG))
          s = tuple(s[j] + row[j] for j in range(NREG))
          m = tuple(jnp.maximum(m[j], row[j]) for j in range(NREG))
        store_acc(s, m)
        return (cur, cnt + jnp.float32(n))

      def rows_slow(iv, base, k0, n, cur, cnt):
        s, m = load_acc()
        ccur, ccnt = cur, cnt
        for k in range(k0, k0 + n):
          d = iv[k]
          roff = (base + k) * EMBED_DIM
          row = tuple(rb[pl.ds(roff + j * NLANE, NLANE)]
                      for j in range(NREG))
          change = d != ccur
          pred = jnp.logical_and(change,
                                 jnp.logical_and(ccur >= d0, ccur < d1))
          flush(pred, ccur, ccnt, s, m)
          s = tuple(jnp.where(change, row[j], s[j] + row[j])
                    for j in range(NREG))
          m = tuple(jnp.where(change, row[j], jnp.maximum(m[j], row[j]))
                    for j in range(NREG))
          ccnt = jnp.where(change, jnp.float32(1.0), ccnt + 1.0)
          ccur = d
        store_acc(s, m)
        return (ccur, ccnt)

      HALF = NLANE // 2

      def group_body(gq, c):
        cur, cnt = c
        iv = ib[pl.ds(gq * NLANE, NLANE)]
        base = gq * NLANE
        u16 = jnp.logical_and(iv[0] == cur, iv[NLANE - 1] == cur)

        def split():
          u0 = jnp.logical_and(iv[0] == cur, iv[HALF - 1] == cur)
          cur1, cnt1 = lax.cond(
              u0,
              lambda: rows_fast(base, HALF, cur, cnt),
              lambda: rows_slow(iv, base, 0, HALF, cur, cnt))
          u1 = jnp.logical_and(iv[HALF] == cur1, iv[NLANE - 1] == cur1)
          return lax.cond(
              u1,
              lambda: rows_fast(base + HALF, HALF, cur1, cnt1),
              lambda: rows_slow(iv, base, HALF, HALF, cur1, cnt1))

        return lax.cond(u16, lambda: rows_fast(base, NLANE, cur, cnt), split)

      return lax.fori_loop(0, CHUNK // NLANE, group_body, (cur0, cnt0))

    issue(jnp.int32(0), rowbuf_a, idxbuf_a, sem_a)

    def pair_body(h, c):
      g0 = 2 * h
      g1 = g0 + 1
      issue(g1, rowbuf_b, idxbuf_b, sem_b)
      wait(g0, rowbuf_a, idxbuf_a, sem_a)
      cur, cnt = process(rowbuf_a, idxbuf_a, c[0], c[1])
      issue(g0 + 2, rowbuf_a, idxbuf_a, sem_a)
      wait(g1, rowbuf_b, idxbuf_b, sem_b)
      return lax.cond(g1 < nch,
                      lambda: process(rowbuf_b, idxbuf_b, cur, cnt),
                      lambda: (cur, cnt))

    cur, cnt = lax.fori_loop(0, (nch + 1) // 2, pair_body,
                             (jnp.int32(-1), jnp.float32(0.0)))
    s_f, m_f = load_acc()
    flush(jnp.logical_and(cur >= d0, cur < d1), cur, cnt, s_f, m_f)

    pltpu.sync_copy(meanbuf, mean_hbm.at[pl.ds(d0 * EMBED_DIM, DPW * EMBED_DIM)])
    pltpu.sync_copy(maxbuf, max_hbm.at[pl.ds(d0 * EMBED_DIM, DPW * EMBED_DIM)])

  return pool_kernel(emb, idx)


def _tc_project(mean_p, max_p, w1t, w2t, b2d):
  RB = NDP // 4

  def mm(mean_ref, max_ref, w1_ref, w2_ref, b_ref, o_ref):
    o_ref[...] = (
        jnp.dot(mean_ref[...], w1_ref[...], preferred_element_type=jnp.float32)
        + jnp.dot(max_ref[...], w2_ref[...], preferred_element_type=jnp.float32)
        + b_ref[...])

  return pl.pallas_call(
      mm,
      grid=(NDP // RB,),
      in_specs=[
          pl.BlockSpec((RB, EMBED_DIM), lambda i: (i, 0)),
          pl.BlockSpec((RB, EMBED_DIM), lambda i: (i, 0)),
          pl.BlockSpec((EMBED_DIM, EMBED_DIM), lambda i: (0, 0)),
          pl.BlockSpec((EMBED_DIM, EMBED_DIM), lambda i: (0, 0)),
          pl.BlockSpec((1, EMBED_DIM), lambda i: (0, 0)),
      ],
      out_specs=pl.BlockSpec((RB, EMBED_DIM), lambda i: (i, 0)),
      out_shape=jax.ShapeDtypeStruct((NDP, EMBED_DIM), jnp.float32),
  )(mean_p, max_p, w1t, w2t, b2d)


def kernel(pulse_embeddings, pulse_to_dom_idx, num_doms, W, b):
  idx = pulse_to_dom_idx.astype(jnp.int32)
  mean_p, max_p = _sc_pool(pulse_embeddings.reshape(-1), idx)
  mean_p = mean_p.reshape(NDP, EMBED_DIM)
  max_p = max_p.reshape(NDP, EMBED_DIM)
  w1t = W[:, :EMBED_DIM].T
  w2t = W[:, EMBED_DIM:].T
  b2d = b.reshape(1, EMBED_DIM)
  out = _tc_project(mean_p, max_p, w1t, w2t, b2d)
  return out[:NUM_DOMS]


# speedup vs baseline: 1.0782x; 1.0782x over previous
import functools

import jax
import jax.numpy as jnp
from jax import lax
from jax.experimental import pallas as pl
from jax.experimental.pallas import tpu as pltpu
from jax.experimental.pallas import tpu_sc as plsc

N_PULSES = 320000
NUM_DOMS = 10000
EMBED_DIM = 128
NLANE = 16
NREG = EMBED_DIM // NLANE

NC, NS = 2, 16
NW = NC * NS
DPW = 8 * -(-NUM_DOMS // (NW * 8))
NDP = DPW * NW
CHUNK = 128
NBLK = N_PULSES // 16
SEARCH_ITERS = 15


def _sc_pool(emb, idx):
  mesh = plsc.VectorSubcoreMesh(core_axis_name="c", subcore_axis_name="s")

  @functools.partial(
      pl.kernel,
      mesh=mesh,
      out_type=[
          jax.ShapeDtypeStruct((NDP * EMBED_DIM,), jnp.float32),
          jax.ShapeDtypeStruct((NDP * EMBED_DIM,), jnp.float32),
      ],
      scratch_types=[
          pltpu.VMEM((CHUNK * EMBED_DIM,), jnp.float32),
          pltpu.VMEM((CHUNK * EMBED_DIM,), jnp.float32),
          pltpu.VMEM((CHUNK,), jnp.int32),
          pltpu.VMEM((CHUNK,), jnp.int32),
          pltpu.VMEM((DPW * EMBED_DIM,), jnp.float32),
          pltpu.VMEM((DPW * EMBED_DIM,), jnp.float32),
          pltpu.VMEM((16,), jnp.int32),
          pltpu.VMEM((16,), jnp.int32),
          pltpu.VMEM((2 * EMBED_DIM,), jnp.float32),
          pltpu.SemaphoreType.DMA,
          pltpu.SemaphoreType.DMA,
      ],
  )
  def pool_kernel(emb_hbm, idx_hbm, mean_hbm, max_hbm,
                  rowbuf_a, rowbuf_b, idxbuf_a, idxbuf_b,
                  meanbuf, maxbuf, sbuf, sbuf2, accbuf, sem_a, sem_b):
    wid = lax.axis_index("s") * NC + lax.axis_index("c")
    d0 = wid * DPW
    d1 = jnp.minimum(d0 + DPW, NUM_DOMS)

    zeros = jnp.zeros((NLANE,), jnp.float32)

    def zero_row(r, carry):
      for j in range(NREG):
        meanbuf[pl.ds(r * EMBED_DIM + j * NLANE, NLANE)] = zeros
        maxbuf[pl.ds(r * EMBED_DIM + j * NLANE, NLANE)] = zeros
      return carry
    lax.fori_loop(0, DPW, zero_row, 0)

    def lower_bound2(t0, t1):
      def step(lo, hi, buf, target):
        probe = buf[pl.ds(0, NLANE)]
        ge = probe[0] >= target
        nlo = jnp.where(ge, lo, (lo + hi) // 2 + 1)
        nhi = jnp.where(ge, (lo + hi) // 2, hi)
        run = lo < hi
        return (jnp.where(run, nlo, lo), jnp.where(run, nhi, hi))

      def it(_, c):
        lo0, hi0, lo1, hi1 = c
        mid0 = jnp.where(lo0 < hi0, (lo0 + hi0) // 2, 0)
        mid1 = jnp.where(lo1 < hi1, (lo1 + hi1) // 2, 0)
        pltpu.make_async_copy(
            idx_hbm.at[pl.ds(mid0 * 16, 16)], sbuf, sem_a).start()
        pltpu.make_async_copy(
            idx_hbm.at[pl.ds(mid1 * 16, 16)], sbuf2, sem_b).start()
        pltpu.make_async_copy(
            idx_hbm.at[pl.ds(0, 16)], sbuf, sem_a).wait()
        pltpu.make_async_copy(
            idx_hbm.at[pl.ds(0, 16)], sbuf2, sem_b).wait()
        lo0, hi0 = step(lo0, hi0, sbuf, t0)
        lo1, hi1 = step(lo1, hi1, sbuf2, t1)
        return (lo0, hi0, lo1, hi1)

      lo0, _, lo1, _ = lax.fori_loop(
          0, SEARCH_ITERS, it,
          (jnp.int32(0), jnp.int32(NBLK), jnp.int32(0), jnp.int32(NBLK)))
      return lo0, lo1

    f0, f1 = lower_bound2(d0, d1)
    p0 = jnp.maximum(f0 - 1, 0) * 16
    p1 = f1 * 16
    c0 = p0 // CHUNK
    nch = (p1 + CHUNK - 1) // CHUNK - c0

    def flush(pred, cur, cnt, s, m):
      def fbody(i, z):
        off = (cur - d0) * EMBED_DIM
        cnt_v = jnp.broadcast_to(cnt, (NLANE,))
        for j in range(NREG):
          meanbuf[pl.ds(off + j * NLANE, NLANE)] = s[j] / cnt_v
          maxbuf[pl.ds(off + j * NLANE, NLANE)] = m[j]
        return z
      lax.fori_loop(0, pred.astype(jnp.int32), fbody, jnp.int32(0))

    NB2 = CHUNK * EMBED_DIM

    def issue(g, rb, ib, sem):
      @pl.when(g < nch)
      def _():
        base = (c0 + g) * CHUNK
        pltpu.make_async_copy(
            emb_hbm.at[pl.ds(base * EMBED_DIM, NB2)], rb, sem).start()
        pltpu.make_async_copy(idx_hbm.at[pl.ds(base, CHUNK)], ib, sem).start()

    def wait(g, rb, ib, sem):
      @pl.when(g < nch)
      def _():
        pltpu.make_async_copy(emb_hbm.at[pl.ds(0, NB2)], rb, sem).wait()
        pltpu.make_async_copy(idx_hbm.at[pl.ds(0, CHUNK)], ib, sem).wait()

    def load_acc():
      sv = tuple(accbuf[pl.ds(j * NLANE, NLANE)] for j in range(NREG))
      mv = tuple(accbuf[pl.ds(EMBED_DIM + j * NLANE, NLANE)]
                 for j in range(NREG))
      return sv, mv

    def store_acc(s, m):
      for j in range(NREG):
        accbuf[pl.ds(j * NLANE, NLANE)] = s[j]
        accbuf[pl.ds(EMBED_DIM + j * NLANE, NLANE)] = m[j]

    def process(rb, ib, cur0, cnt0):
      def rows_fast(row0, n, cur, cnt):
        s, m = load_acc()
        for k in range(n):
          roff = (row0 + k) * EMBED_DIM
          row = tuple(rb[pl.ds(roff + j * NLANE, NLANE)]
                      for j in range(NREG))
          s = tuple(s[j] + row[j] for j in range(NREG))
          m = tuple(jnp.maximum(m[j], row[j]) for j in range(NREG))
        store_acc(s, m)
        return (cur, cnt + jnp.float32(n))

      def rows_slow(iv, base, k0, n, cur, cnt):
        s, m = load_acc()
        ccur, ccnt = cur, cnt
        for k in range(k0, k0 + n):
          d = iv[k]
          roff = (base + k) * EMBED_DIM
          row = tuple(rb[pl.ds(roff + j * NLANE, NLANE)]
                      for j in range(NREG))
          change = d != ccur
          pred = jnp.logical_and(change,
                                 jnp.logical_and(ccur >= d0, ccur < d1))
          flush(pred, ccur, ccnt, s, m)
          s = tuple(jnp.where(change, row[j], s[j] + row[j])
                    for j in range(NREG))
          m = tuple(jnp.where(change, row[j], jnp.maximum(m[j], row[j]))
                    for j in range(NREG))
          ccnt = jnp.where(change, jnp.float32(1.0), ccnt + 1.0)
          ccur = d
        store_acc(s, m)
        return (ccur, ccnt)

      def group_body(gq, c):
        cur, cnt = c
        iv = ib[pl.ds(gq * NLANE, NLANE)]
        base = gq * NLANE
        u16 = jnp.logical_and(iv[0] == cur, iv[NLANE - 1] == cur)
        return lax.cond(u16,
                        lambda: rows_fast(base, NLANE, cur, cnt),
                        lambda: rows_slow(iv, base, 0, NLANE, cur, cnt))

      return lax.fori_loop(0, CHUNK // NLANE, group_body, (cur0, cnt0))

    issue(jnp.int32(0), rowbuf_a, idxbuf_a, sem_a)

    def pair_body(h, c):
      g0 = 2 * h
      g1 = g0 + 1
      issue(g1, rowbuf_b, idxbuf_b, sem_b)
      wait(g0, rowbuf_a, idxbuf_a, sem_a)
      cur, cnt = process(rowbuf_a, idxbuf_a, c[0], c[1])
      issue(g0 + 2, rowbuf_a, idxbuf_a, sem_a)
      wait(g1, rowbuf_b, idxbuf_b, sem_b)
      return lax.cond(g1 < nch,
                      lambda: process(rowbuf_b, idxbuf_b, cur, cnt),
                      lambda: (cur, cnt))

    cur, cnt = lax.fori_loop(0, (nch + 1) // 2, pair_body,
                             (jnp.int32(-1), jnp.float32(0.0)))
    s_f, m_f = load_acc()
    flush(jnp.logical_and(cur >= d0, cur < d1), cur, cnt, s_f, m_f)

    out_sl = pl.ds(d0 * EMBED_DIM, DPW * EMBED_DIM)
    pltpu.make_async_copy(meanbuf, mean_hbm.at[out_sl], sem_a).start()
    pltpu.make_async_copy(maxbuf, max_hbm.at[out_sl], sem_b).start()
    pltpu.make_async_copy(meanbuf, mean_hbm.at[out_sl], sem_a).wait()
    pltpu.make_async_copy(maxbuf, max_hbm.at[out_sl], sem_b).wait()

  return pool_kernel(emb, idx)


def _tc_project(mean_p, max_p, w1t, w2t, b2d):
  RB = NDP // 4

  def mm(mean_ref, max_ref, w1_ref, w2_ref, b_ref, o_ref):
    o_ref[...] = (
        jnp.dot(mean_ref[...], w1_ref[...], preferred_element_type=jnp.float32)
        + jnp.dot(max_ref[...], w2_ref[...], preferred_element_type=jnp.float32)
        + b_ref[...])

  return pl.pallas_call(
      mm,
      grid=(NDP // RB,),
      in_specs=[
          pl.BlockSpec((RB, EMBED_DIM), lambda i: (i, 0)),
          pl.BlockSpec((RB, EMBED_DIM), lambda i: (i, 0)),
          pl.BlockSpec((EMBED_DIM, EMBED_DIM), lambda i: (0, 0)),
          pl.BlockSpec((EMBED_DIM, EMBED_DIM), lambda i: (0, 0)),
          pl.BlockSpec((1, EMBED_DIM), lambda i: (0, 0)),
      ],
      out_specs=pl.BlockSpec((RB, EMBED_DIM), lambda i: (i, 0)),
      out_shape=jax.ShapeDtypeStruct((NDP, EMBED_DIM), jnp.float32),
  )(mean_p, max_p, w1t, w2t, b2d)


def kernel(pulse_embeddings, pulse_to_dom_idx, num_doms, W, b):
  idx = pulse_to_dom_idx.astype(jnp.int32)
  mean_p, max_p = _sc_pool(pulse_embeddings.reshape(-1), idx)
  mean_p = mean_p.reshape(NDP, EMBED_DIM)
  max_p = max_p.reshape(NDP, EMBED_DIM)
  w1t = W[:, :EMBED_DIM].T
  w2t = W[:, EMBED_DIM:].T
  b2d = b.reshape(1, EMBED_DIM)
  out = _tc_project(mean_p, max_p, w1t, w2t, b2d)
  return out[:NUM_DOMS]
